# baseline (speedup 1.0000x reference)
import functools

import jax
import jax.numpy as jnp
from jax import lax
from jax.experimental import pallas as pl
from jax.experimental.pallas import tpu as pltpu
from jax.experimental.pallas import tpu_sc as plsc

SEQ = 8192
D = 768
EPS = 1e-12

CH_SIZES = (2048, 2048, 4096)
CH_STARTS = (0, 2048, 4096)
P = len(CH_SIZES)

NC = 2
NS = 16
NW = NC * NS
CHUNK = 64


def _sc_body(ids_hbm, table_hbm, out_hbm, idx_v, rows_v,
             wsem0, wsem1, osem0, osem1, *, nch, rpw):
    wid = lax.axis_index("s") * NC + lax.axis_index("c")
    base = wid * rpw
    pltpu.sync_copy(ids_hbm.at[wid], idx_v)
    wsems = (wsem0, wsem1)
    osems = (osem0, osem1)
    gathers = [
        pltpu.async_copy(table_hbm.at[idx_v.at[0]], rows_v.at[0], wsem0)
    ]
    outs = []
    for c in range(nch):
        b = c % 2
        if c + 1 < nch:
            if c >= 1:
                outs[c - 1].wait()
            gathers.append(
                pltpu.async_copy(table_hbm.at[idx_v.at[c + 1]],
                                 rows_v.at[(c + 1) % 2], wsems[(c + 1) % 2]))
        gathers[c].wait()
        outs.append(
            pltpu.async_copy(
                rows_v.at[b], out_hbm.at[pl.ds(base + c * CHUNK, CHUNK)],
                osems[b]))
    outs[-1].wait()
    if nch > 1:
        outs[-2].wait()


@functools.lru_cache(maxsize=None)
def _make_sc_gather(size):
    rpw = size // NW
    nch = rpw // CHUNK
    mesh = plsc.VectorSubcoreMesh(core_axis_name="c", subcore_axis_name="s")
    return functools.partial(
        pl.kernel,
        out_type=jax.ShapeDtypeStruct((size, D), jnp.float32),
        mesh=mesh,
        scratch_types=[
            pltpu.VMEM((nch, CHUNK), jnp.int32),
            pltpu.VMEM((2, CHUNK, D), jnp.float32),
            pltpu.SemaphoreType.DMA,
            pltpu.SemaphoreType.DMA,
            pltpu.SemaphoreType.DMA,
            pltpu.SemaphoreType.DMA,
        ],
    )(functools.partial(_sc_body, nch=nch, rpw=rpw))


BT = 2048


def _tc_body_first(word_ref, pos_ref, tt_ref, tok_ref, gam_ref, bet_ref,
                   w_ref, b_ref, out_ref):
    x = word_ref[...] + pos_ref[...]
    t = tt_ref[...].astype(jnp.float32)
    tok0 = tok_ref[0:1, :]
    tok1 = tok_ref[1:2, :]
    x = x + tok0 + t * (tok1 - tok0)
    s1 = jnp.sum(x, axis=-1, keepdims=True)
    s2 = jnp.sum(x * x, axis=-1, keepdims=True)
    mean = s1 * (1.0 / D)
    var = s2 * (1.0 / D) - mean * mean
    y = (x - mean) * lax.rsqrt(var + EPS) * gam_ref[...] + bet_ref[...]
    acc = lax.dot_general(
        y.astype(jnp.bfloat16), w_ref[...],
        dimension_numbers=(((1,), (1,)), ((), ())),
        preferred_element_type=jnp.float32,
    )
    out_ref[...] = acc + b_ref[...]


def _tc_body_chained(word_ref, pos_ref, tt_ref, tok_ref, gam_ref, bet_ref,
                     w_ref, b_ref, _buf_ref, out_ref):
    _tc_body_first(word_ref, pos_ref, tt_ref, tok_ref, gam_ref, bet_ref,
                   w_ref, b_ref, out_ref)


def _common_in_specs(b0):
    return [
        pl.BlockSpec((BT, D), lambda i: (i, 0)),
        pl.BlockSpec((BT, D), lambda i, b0=b0: (b0 + i, 0)),
        pl.BlockSpec((BT, 1), lambda i, b0=b0: (b0 + i, 0)),
        pl.BlockSpec((2, D), lambda i: (0, 0)),
        pl.BlockSpec((1, D), lambda i: (0, 0)),
        pl.BlockSpec((1, D), lambda i: (0, 0)),
        pl.BlockSpec((D, D), lambda i: (0, 0)),
        pl.BlockSpec((1, D), lambda i: (0, 0)),
    ]


@functools.lru_cache(maxsize=None)
def _make_tc_call(start, size, first):
    b0 = start // BT
    nbt = size // BT
    out_spec = pl.BlockSpec((BT, D), lambda i, b0=b0: (b0 + i, 0))
    out_shape = jax.ShapeDtypeStruct((SEQ, D), jnp.float32)
    if first:
        return pl.pallas_call(
            _tc_body_first,
            grid=(nbt,),
            in_specs=_common_in_specs(b0),
            out_specs=out_spec,
            out_shape=out_shape,
        )
    return pl.pallas_call(
        _tc_body_chained,
        grid=(nbt,),
        in_specs=_common_in_specs(b0)
        + [pl.BlockSpec(memory_space=pl.ANY)],
        out_specs=out_spec,
        out_shape=out_shape,
        input_output_aliases={8: 0},
    )


def kernel(input_ids, token_type_ids, word_table, pos_table, tok_table,
           ln_gamma, ln_beta, W_lin, b_lin):
    tt2 = token_type_ids.reshape(SEQ, 1)
    gam = ln_gamma.reshape(1, D)
    bet = ln_beta.reshape(1, D)
    wb = W_lin.astype(jnp.bfloat16)
    b2 = b_lin.reshape(1, D)

    chunks = []
    for start, size in zip(CH_STARTS, CH_SIZES):
        ids = input_ids[start:start + size].reshape(NW, size // NW // CHUNK,
                                                    CHUNK)
        chunks.append(_make_sc_gather(size)(ids, word_table))
    buf = None
    for p, (start, size) in enumerate(zip(CH_STARTS, CH_SIZES)):
        args = (chunks[p], pos_table[:SEQ], tt2, tok_table, gam, bet, wb, b2)
        if p == 0:
            buf = _make_tc_call(start, size, True)(*args)
        else:
            buf = _make_tc_call(start, size, False)(*args, buf)
    return buf.reshape(1, SEQ, D)


# speedup vs baseline: 1.1022x; 1.1022x over previous
import functools

import jax
import jax.numpy as jnp
from jax import lax
from jax.experimental import pallas as pl
from jax.experimental.pallas import tpu as pltpu
from jax.experimental.pallas import tpu_sc as plsc

SEQ = 8192
D = 768
EPS = 1e-12

CH_SIZES = (4096, 4096)
CH_STARTS = (0, 4096)
P = len(CH_SIZES)

NC = 2
NS = 16
NW = NC * NS
CHUNK = 64


def _sc_body(ids_hbm, table_hbm, out_hbm, idx_v, rows_v,
             wsem0, wsem1, osem0, osem1, *, nch, rpw):
    wid = lax.axis_index("s") * NC + lax.axis_index("c")
    base = wid * rpw
    pltpu.sync_copy(ids_hbm.at[wid], idx_v)
    wsems = (wsem0, wsem1)
    osems = (osem0, osem1)
    gathers = [
        pltpu.async_copy(table_hbm.at[idx_v.at[0]], rows_v.at[0], wsem0)
    ]
    outs = []
    for c in range(nch):
        b = c % 2
        if c + 1 < nch:
            if c >= 1:
                outs[c - 1].wait()
            gathers.append(
                pltpu.async_copy(table_hbm.at[idx_v.at[c + 1]],
                                 rows_v.at[(c + 1) % 2], wsems[(c + 1) % 2]))
        gathers[c].wait()
        outs.append(
            pltpu.async_copy(
                rows_v.at[b], out_hbm.at[pl.ds(base + c * CHUNK, CHUNK)],
                osems[b]))
    outs[-1].wait()
    if nch > 1:
        outs[-2].wait()


@functools.lru_cache(maxsize=None)
def _make_sc_gather(size):
    rpw = size // NW
    nch = rpw // CHUNK
    mesh = plsc.VectorSubcoreMesh(core_axis_name="c", subcore_axis_name="s")
    return functools.partial(
        pl.kernel,
        out_type=jax.ShapeDtypeStruct((size, D), jnp.float32),
        mesh=mesh,
        scratch_types=[
            pltpu.VMEM((nch, CHUNK), jnp.int32),
            pltpu.VMEM((2, CHUNK, D), jnp.float32),
            pltpu.SemaphoreType.DMA,
            pltpu.SemaphoreType.DMA,
            pltpu.SemaphoreType.DMA,
            pltpu.SemaphoreType.DMA,
        ],
    )(functools.partial(_sc_body, nch=nch, rpw=rpw))


BT = 2048


def _tc_body_first(word_ref, pos_ref, tt_ref, tok_ref, gam_ref, bet_ref,
                   w_ref, b_ref, out_ref):
    x = word_ref[...] + pos_ref[...]
    t = tt_ref[...].astype(jnp.float32)
    tok0 = tok_ref[0:1, :]
    tok1 = tok_ref[1:2, :]
    x = x + tok0 + t * (tok1 - tok0)
    s1 = jnp.sum(x, axis=-1, keepdims=True)
    s2 = jnp.sum(x * x, axis=-1, keepdims=True)
    mean = s1 * (1.0 / D)
    var = s2 * (1.0 / D) - mean * mean
    y = (x - mean) * lax.rsqrt(var + EPS) * gam_ref[...] + bet_ref[...]
    acc = lax.dot_general(
        y.astype(jnp.bfloat16), w_ref[...],
        dimension_numbers=(((1,), (1,)), ((), ())),
        preferred_element_type=jnp.float32,
    )
    out_ref[...] = acc + b_ref[...]


def _tc_body_chained(word_ref, pos_ref, tt_ref, tok_ref, gam_ref, bet_ref,
                     w_ref, b_ref, _buf_ref, out_ref):
    _tc_body_first(word_ref, pos_ref, tt_ref, tok_ref, gam_ref, bet_ref,
                   w_ref, b_ref, out_ref)


def _common_in_specs(b0):
    return [
        pl.BlockSpec((BT, D), lambda i: (i, 0)),
        pl.BlockSpec((BT, D), lambda i, b0=b0: (b0 + i, 0)),
        pl.BlockSpec((BT, 1), lambda i, b0=b0: (b0 + i, 0)),
        pl.BlockSpec((2, D), lambda i: (0, 0)),
        pl.BlockSpec((1, D), lambda i: (0, 0)),
        pl.BlockSpec((1, D), lambda i: (0, 0)),
        pl.BlockSpec((D, D), lambda i: (0, 0)),
        pl.BlockSpec((1, D), lambda i: (0, 0)),
    ]


@functools.lru_cache(maxsize=None)
def _make_tc_call(start, size, first):
    b0 = start // BT
    nbt = size // BT
    out_spec = pl.BlockSpec((BT, D), lambda i, b0=b0: (b0 + i, 0))
    out_shape = jax.ShapeDtypeStruct((SEQ, D), jnp.float32)
    if first:
        return pl.pallas_call(
            _tc_body_first,
            grid=(nbt,),
            in_specs=_common_in_specs(b0),
            out_specs=out_spec,
            out_shape=out_shape,
        )
    return pl.pallas_call(
        _tc_body_chained,
        grid=(nbt,),
        in_specs=_common_in_specs(b0)
        + [pl.BlockSpec(memory_space=pl.ANY)],
        out_specs=out_spec,
        out_shape=out_shape,
        input_output_aliases={8: 0},
    )


def kernel(input_ids, token_type_ids, word_table, pos_table, tok_table,
           ln_gamma, ln_beta, W_lin, b_lin):
    tt2 = token_type_ids.reshape(SEQ, 1)
    gam = ln_gamma.reshape(1, D)
    bet = ln_beta.reshape(1, D)
    wb = W_lin.astype(jnp.bfloat16)
    b2 = b_lin.reshape(1, D)

    chunks = []
    for start, size in zip(CH_STARTS, CH_SIZES):
        ids = input_ids[start:start + size].reshape(NW, size // NW // CHUNK,
                                                    CHUNK)
        chunks.append(_make_sc_gather(size)(ids, word_table))
    buf = None
    for p, (start, size) in enumerate(zip(CH_STARTS, CH_SIZES)):
        args = (chunks[p], pos_table[:SEQ], tt2, tok_table, gam, bet, wb, b2)
        if p == 0:
            buf = _make_tc_call(start, size, True)(*args)
        else:
            buf = _make_tc_call(start, size, False)(*args, buf)
    return buf.reshape(1, SEQ, D)
